# Initial kernel scaffold; baseline (speedup 1.0000x reference)
import jax
import jax.numpy as jnp
from jax.experimental import pallas as pl


def kernel(x, expert_indices, expert_weights, w1_stacked, w2_stacked, w3_stacked):
    raise NotImplementedError("write your pallas kernel here")


# speedup vs baseline: 4.0983x; 4.0983x over previous
import functools

import jax
import jax.numpy as jnp
from jax.experimental import pallas as pl
from jax.experimental.pallas import tpu as pltpu


def _moe_body(eids_ref, pairs_ref, wsort_ref, x_ref, w1_ref, w3_ref, w2_ref,
              out_ref, *, top_k):
    i = pl.program_id(0)

    @pl.when(i == 0)
    def _init():
        out_ref[...] = jnp.zeros_like(out_ref)

    p = pairs_ref[i]
    t = p // top_k
    xrow = x_ref[pl.ds(t, 1), :]
    g = jnp.dot(xrow, w1_ref[0], preferred_element_type=jnp.float32)
    u = jnp.dot(xrow, w3_ref[0], preferred_element_type=jnp.float32)
    h = (g * jax.nn.sigmoid(g)) * u
    o = jnp.dot(h, w2_ref[0], preferred_element_type=jnp.float32)
    w = wsort_ref[i]
    out_ref[pl.ds(t, 1), :] += w * o


def kernel(x, expert_indices, expert_weights, w1_stacked, w2_stacked, w3_stacked):
    B, H = x.shape
    K = expert_indices.shape[1]
    E, _, I = w1_stacked.shape
    P = B * K

    eids = expert_indices.reshape(P).astype(jnp.int32)
    order = jnp.argsort(eids).astype(jnp.int32)
    sorted_eids = eids[order]
    sorted_w = expert_weights.reshape(P)[order]

    grid_spec = pltpu.PrefetchScalarGridSpec(
        num_scalar_prefetch=3,
        grid=(P,),
        in_specs=[
            pl.BlockSpec((B, H), lambda i, e, p, w: (0, 0)),
            pl.BlockSpec((1, H, I), lambda i, e, p, w: (e[i], 0, 0)),
            pl.BlockSpec((1, H, I), lambda i, e, p, w: (e[i], 0, 0)),
            pl.BlockSpec((1, I, H), lambda i, e, p, w: (e[i], 0, 0)),
        ],
        out_specs=pl.BlockSpec((B, H), lambda i, e, p, w: (0, 0)),
    )
    fn = pl.pallas_call(
        functools.partial(_moe_body, top_k=K),
        grid_spec=grid_spec,
        out_shape=jax.ShapeDtypeStruct((B, H), jnp.float32),
    )
    return fn(sorted_eids, order, sorted_w, x, w1_stacked, w3_stacked, w2_stacked)
